# baseline (speedup 1.0000x reference)
import functools

import jax
import jax.numpy as jnp
from jax.experimental import pallas as pl

_PREC = jax.lax.Precision.HIGHEST


def _conv_body(x_ref, w_ref, b_ref, o_ref, *, m, wpp, cin, cout):
    acc = None
    for dh in range(3):
        for dw in range(3):
            s = x_ref[0, pl.ds(dh * wpp + dw, m), :]
            p = jnp.dot(s, w_ref[dh * 3 + dw],
                        preferred_element_type=jnp.float32, precision=_PREC)
            acc = p if acc is None else acc + p
    o_ref[0] = jnp.maximum(acc + b_ref[...], 0.0)


def _conv(x, wt, b, th, edge, wp=None):
    h, w, cin = x.shape
    cout = wt.shape[2]
    wp = wp or w
    wpp = wp + 2
    nt = h // th
    m = th * wpp
    xp = jnp.pad(x, ((1, 1), (1, 1), (0, 0)), mode='edge' if edge else 'constant')
    if wp > w:
        xp = jnp.pad(xp, ((0, 0), (0, wp - w), (0, 0)))
    r = (th + 2) * wpp + 8
    xt = jnp.stack([jax.lax.slice_in_dim(xp, i * th, i * th + th + 2, axis=0)
                    for i in range(nt)]).reshape(nt, (th + 2) * wpp, cin)
    xt = jnp.pad(xt, ((0, 0), (0, 8), (0, 0)))
    out = pl.pallas_call(
        functools.partial(_conv_body, m=m, wpp=wpp, cin=cin, cout=cout),
        grid=(nt,),
        in_specs=[
            pl.BlockSpec((1, r, cin), lambda i: (i, 0, 0)),
            pl.BlockSpec((9, cin, cout), lambda i: (0, 0, 0)),
            pl.BlockSpec((1, cout), lambda i: (0, 0)),
        ],
        out_specs=pl.BlockSpec((1, m, cout), lambda i: (i, 0, 0)),
        out_shape=jax.ShapeDtypeStruct((nt, m, cout), jnp.float32),
    )(xt, wt, b.reshape(1, cout))
    return out.reshape(h, wpp, cout)[:, :w, :]


def _pool_body(a_ref, b_ref, c_ref, d_ref, o_ref):
    o_ref[...] = jnp.maximum(jnp.maximum(a_ref[...], b_ref[...]),
                             jnp.maximum(c_ref[...], d_ref[...]))


def _pool(x, th2):
    h, w, c = x.shape
    h2, w2 = h // 2, w // 2
    views = [x[i::2, j::2, :].reshape(h2, w2 * c) for i in (0, 1) for j in (0, 1)]
    nt = h2 // th2
    spec = pl.BlockSpec((th2, w2 * c), lambda i: (i, 0))
    out = pl.pallas_call(
        _pool_body,
        grid=(nt,),
        in_specs=[spec] * 4,
        out_specs=spec,
        out_shape=jax.ShapeDtypeStruct((h2, w2 * c), jnp.float32),
    )(*views)
    return out.reshape(h2, w2, c)


def kernel(batch, pooling_mask, c1_w, c1_b, c2_w, c2_b, c3_w, c3_b, c4_w, c4_b,
           W5, b5, W6, b6, W7, b7, W8, b8, W9, b9, W10, b10):
    x = batch[0].transpose(1, 2, 0)
    x = _conv(x, c1_w.transpose(2, 3, 1, 0).reshape(9, 3, 64), c1_b, th=28, edge=False)
    x = _conv(x, c2_w.transpose(2, 3, 1, 0).reshape(9, 64, 64), c2_b, th=28, edge=False)
    x = _pool(x, th2=16)
    x = _conv(x, c3_w.transpose(2, 3, 1, 0).reshape(9, 64, 128), c3_b, th=28, edge=False)
    x = _conv(x, c4_w.transpose(2, 3, 1, 0).reshape(9, 128, 128), c4_b, th=28, edge=False)
    x = _pool(x, th2=8)
    x = _conv(x, W5.reshape(9, 128, 256), b5, th=28, edge=True)
    x = _conv(x, W6.reshape(9, 256, 256), b6, th=28, edge=True)
    x = _conv(x, W7.reshape(9, 256, 256), b7, th=28, edge=True)
    x = _pool(x, th2=28)
    x = _conv(x, W8.reshape(9, 256, 512), b8, th=28, edge=True, wp=32)
    x = _conv(x, W9.reshape(9, 512, 512), b9, th=28, edge=True, wp=32)
    x = _conv(x, W10.reshape(9, 512, 512), b10, th=28, edge=True, wp=32)
    return x.transpose(2, 0, 1)[None]


# speedup vs baseline: 1.9313x; 1.9313x over previous
import functools

import jax
import jax.numpy as jnp
from jax.experimental import pallas as pl

_PREC = jax.lax.Precision.DEFAULT


def _conv_body(x_ref, w_ref, b_ref, o_ref, *, m, wpp, cin, cout):
    acc = None
    for dh in range(3):
        for dw in range(3):
            s = x_ref[0, pl.ds(dh * wpp + dw, m), :]
            p = jnp.dot(s, w_ref[dh * 3 + dw],
                        preferred_element_type=jnp.float32, precision=_PREC)
            acc = p if acc is None else acc + p
    o_ref[0] = jnp.maximum(acc + b_ref[...], 0.0)


def _conv(x, wt, b, th, edge, wp=None):
    h, w, cin = x.shape
    cout = wt.shape[2]
    wp = wp or w
    wpp = wp + 2
    nt = h // th
    m = th * wpp
    xp = jnp.pad(x, ((1, 1), (1, 1), (0, 0)), mode='edge' if edge else 'constant')
    if wp > w:
        xp = jnp.pad(xp, ((0, 0), (0, wp - w), (0, 0)))
    r = (th + 2) * wpp + 8
    xt = jnp.stack([jax.lax.slice_in_dim(xp, i * th, i * th + th + 2, axis=0)
                    for i in range(nt)]).reshape(nt, (th + 2) * wpp, cin)
    xt = jnp.pad(xt, ((0, 0), (0, 8), (0, 0)))
    out = pl.pallas_call(
        functools.partial(_conv_body, m=m, wpp=wpp, cin=cin, cout=cout),
        grid=(nt,),
        in_specs=[
            pl.BlockSpec((1, r, cin), lambda i: (i, 0, 0)),
            pl.BlockSpec((9, cin, cout), lambda i: (0, 0, 0)),
            pl.BlockSpec((1, cout), lambda i: (0, 0)),
        ],
        out_specs=pl.BlockSpec((1, m, cout), lambda i: (i, 0, 0)),
        out_shape=jax.ShapeDtypeStruct((nt, m, cout), jnp.float32),
    )(xt, wt, b.reshape(1, cout))
    return out.reshape(h, wpp, cout)[:, :w, :]


def _pool_body(a_ref, b_ref, c_ref, d_ref, o_ref):
    o_ref[...] = jnp.maximum(jnp.maximum(a_ref[...], b_ref[...]),
                             jnp.maximum(c_ref[...], d_ref[...]))


def _pool(x, th2):
    h, w, c = x.shape
    h2, w2 = h // 2, w // 2
    views = [x[i::2, j::2, :].reshape(h2, w2 * c) for i in (0, 1) for j in (0, 1)]
    nt = h2 // th2
    spec = pl.BlockSpec((th2, w2 * c), lambda i: (i, 0))
    out = pl.pallas_call(
        _pool_body,
        grid=(nt,),
        in_specs=[spec] * 4,
        out_specs=spec,
        out_shape=jax.ShapeDtypeStruct((h2, w2 * c), jnp.float32),
    )(*views)
    return out.reshape(h2, w2, c)


def kernel(batch, pooling_mask, c1_w, c1_b, c2_w, c2_b, c3_w, c3_b, c4_w, c4_b,
           W5, b5, W6, b6, W7, b7, W8, b8, W9, b9, W10, b10):
    x = batch[0].transpose(1, 2, 0)
    x = _conv(x, c1_w.transpose(2, 3, 1, 0).reshape(9, 3, 64), c1_b, th=28, edge=False)
    x = _conv(x, c2_w.transpose(2, 3, 1, 0).reshape(9, 64, 64), c2_b, th=28, edge=False)
    x = _pool(x, th2=16)
    x = _conv(x, c3_w.transpose(2, 3, 1, 0).reshape(9, 64, 128), c3_b, th=28, edge=False)
    x = _conv(x, c4_w.transpose(2, 3, 1, 0).reshape(9, 128, 128), c4_b, th=28, edge=False)
    x = _pool(x, th2=8)
    x = _conv(x, W5.reshape(9, 128, 256), b5, th=28, edge=True)
    x = _conv(x, W6.reshape(9, 256, 256), b6, th=28, edge=True)
    x = _conv(x, W7.reshape(9, 256, 256), b7, th=28, edge=True)
    x = _pool(x, th2=28)
    x = _conv(x, W8.reshape(9, 256, 512), b8, th=28, edge=True, wp=32)
    x = _conv(x, W9.reshape(9, 512, 512), b9, th=28, edge=True, wp=32)
    x = _conv(x, W10.reshape(9, 512, 512), b10, th=28, edge=True, wp=32)
    return x.transpose(2, 0, 1)[None]


# speedup vs baseline: 3.3568x; 1.7381x over previous
import functools

import jax
import jax.numpy as jnp
from jax.experimental import pallas as pl


def _conv_body(x_ref, w_ref, b_ref, o_ref, *, m, wp, out_dtype):
    acc = None
    for dh in range(3):
        s = x_ref[0, pl.ds(dh * wp, m), :]
        p = jnp.dot(s, w_ref[dh], preferred_element_type=jnp.float32)
        acc = p if acc is None else acc + p
    o_ref[0] = jnp.maximum(acc + b_ref[...], 0.0).astype(out_dtype)


def _conv(x, wt, b, th, edge, wp=None, out_dtype=jnp.bfloat16):
    h, w, cin = x.shape
    cout = wt.shape[2]
    wp = wp or w
    nt = h // th
    m = th * wp
    xp = jnp.pad(x, ((1, 1), (1, 1), (0, 0)), mode='edge' if edge else 'constant')
    xcat = jnp.concatenate([xp[:, d:d + w, :] for d in range(3)], axis=2)
    if wp > w:
        xcat = jnp.pad(xcat, ((0, 0), (0, wp - w), (0, 0)))
    xt = jnp.stack([jax.lax.slice_in_dim(xcat, i * th, i * th + th + 2, axis=0)
                    for i in range(nt)]).reshape(nt, (th + 2) * wp, 3 * cin)
    out = pl.pallas_call(
        functools.partial(_conv_body, m=m, wp=wp, out_dtype=out_dtype),
        grid=(nt,),
        in_specs=[
            pl.BlockSpec((1, (th + 2) * wp, 3 * cin), lambda i: (i, 0, 0)),
            pl.BlockSpec((3, 3 * cin, cout), lambda i: (0, 0, 0)),
            pl.BlockSpec((1, cout), lambda i: (0, 0)),
        ],
        out_specs=pl.BlockSpec((1, m, cout), lambda i: (i, 0, 0)),
        out_shape=jax.ShapeDtypeStruct((nt, m, cout), out_dtype),
    )(xt, wt, b.reshape(1, cout))
    return out.reshape(h, wp, cout)[:, :w, :]


def _pool_body(a_ref, b_ref, c_ref, d_ref, o_ref):
    o_ref[...] = jnp.maximum(jnp.maximum(a_ref[...], b_ref[...]),
                             jnp.maximum(c_ref[...], d_ref[...]))


def _pool(x, th2):
    h, w, c = x.shape
    h2, w2 = h // 2, w // 2
    views = [x[i::2, j::2, :].reshape(h2, w2 * c) for i in (0, 1) for j in (0, 1)]
    nt = h2 // th2
    spec = pl.BlockSpec((th2, w2 * c), lambda i: (i, 0))
    out = pl.pallas_call(
        _pool_body,
        grid=(nt,),
        in_specs=[spec] * 4,
        out_specs=spec,
        out_shape=jax.ShapeDtypeStruct((h2, w2 * c), x.dtype),
    )(*views)
    return out.reshape(h2, w2, c)


def _w(wt):
    return wt.reshape(3, -1, wt.shape[2]).astype(jnp.bfloat16)


def kernel(batch, pooling_mask, c1_w, c1_b, c2_w, c2_b, c3_w, c3_b, c4_w, c4_b,
           W5, b5, W6, b6, W7, b7, W8, b8, W9, b9, W10, b10):
    bf = jnp.bfloat16
    x = batch[0].transpose(1, 2, 0).astype(bf)
    x = _conv(x, _w(c1_w.transpose(2, 3, 1, 0).reshape(9, 3, 64)), c1_b, th=28, edge=False)
    x = _conv(x, _w(c2_w.transpose(2, 3, 1, 0).reshape(9, 64, 64)), c2_b, th=28, edge=False)
    x = _pool(x, th2=16)
    x = _conv(x, _w(c3_w.transpose(2, 3, 1, 0).reshape(9, 64, 128)), c3_b, th=28, edge=False)
    x = _conv(x, _w(c4_w.transpose(2, 3, 1, 0).reshape(9, 128, 128)), c4_b, th=28, edge=False)
    x = _pool(x, th2=56)
    x = _conv(x, _w(W5.reshape(9, 128, 256)), b5, th=28, edge=True, wp=64)
    x = _conv(x, _w(W6.reshape(9, 256, 256)), b6, th=28, edge=True, wp=64)
    x = _conv(x, _w(W7.reshape(9, 256, 256)), b7, th=28, edge=True, wp=64)
    x = _pool(x, th2=28)
    x = _conv(x, _w(W8.reshape(9, 256, 512)), b8, th=28, edge=True, wp=32)
    x = _conv(x, _w(W9.reshape(9, 512, 512)), b9, th=28, edge=True, wp=32)
    x = _conv(x, _w(W10.reshape(9, 512, 512)), b10, th=28, edge=True, wp=32,
              out_dtype=jnp.float32)
    return x.transpose(2, 0, 1)[None]


# speedup vs baseline: 3.4741x; 1.0349x over previous
import functools

import jax
import jax.numpy as jnp
from jax.experimental import pallas as pl


def _conv_body(x_ref, w_ref, b_ref, o_ref, *, m, wp, out_dtype):
    acc = None
    for dh in range(3):
        s = x_ref[0, pl.ds(dh * wp, m), :]
        p = jnp.dot(s, w_ref[dh], preferred_element_type=jnp.float32)
        acc = p if acc is None else acc + p
    o_ref[0] = jnp.maximum(acc + b_ref[...], 0.0).astype(out_dtype)


def _conv(x, wt, b, th, edge, wp=None, out_dtype=jnp.bfloat16):
    h, w, cin = x.shape
    cout = wt.shape[2]
    wp = wp or w
    th = th or h
    nt = h // th
    m = th * wp
    xp = jnp.pad(x, ((1, 1), (1, 1), (0, 0)), mode='edge' if edge else 'constant')
    xcat = jnp.concatenate([xp[:, d:d + w, :] for d in range(3)], axis=2)
    if wp > w:
        xcat = jnp.pad(xcat, ((0, 0), (0, wp - w), (0, 0)))
    xt = jnp.stack([jax.lax.slice_in_dim(xcat, i * th, i * th + th + 2, axis=0)
                    for i in range(nt)]).reshape(nt, (th + 2) * wp, 3 * cin)
    out = pl.pallas_call(
        functools.partial(_conv_body, m=m, wp=wp, out_dtype=out_dtype),
        grid=(nt,),
        in_specs=[
            pl.BlockSpec((1, (th + 2) * wp, 3 * cin), lambda i: (i, 0, 0)),
            pl.BlockSpec((3, 3 * cin, cout), lambda i: (0, 0, 0)),
            pl.BlockSpec((1, cout), lambda i: (0, 0)),
        ],
        out_specs=pl.BlockSpec((1, m, cout), lambda i: (i, 0, 0)),
        out_shape=jax.ShapeDtypeStruct((nt, m, cout), out_dtype),
    )(xt, wt, b.reshape(1, cout))
    return out.reshape(h, wp, cout)[:, :w, :]


def _pool_body(a_ref, b_ref, c_ref, d_ref, o_ref):
    o_ref[...] = jnp.maximum(jnp.maximum(a_ref[...], b_ref[...]),
                             jnp.maximum(c_ref[...], d_ref[...]))


def _pool(x, th2):
    h, w, c = x.shape
    h2, w2 = h // 2, w // 2
    views = [x[i::2, j::2, :].reshape(h2, w2 * c) for i in (0, 1) for j in (0, 1)]
    nt = h2 // th2
    spec = pl.BlockSpec((th2, w2 * c), lambda i: (i, 0))
    out = pl.pallas_call(
        _pool_body,
        grid=(nt,),
        in_specs=[spec] * 4,
        out_specs=spec,
        out_shape=jax.ShapeDtypeStruct((h2, w2 * c), x.dtype),
    )(*views)
    return out.reshape(h2, w2, c)


def _w(wt):
    return wt.reshape(3, -1, wt.shape[2]).astype(jnp.bfloat16)


def kernel(batch, pooling_mask, c1_w, c1_b, c2_w, c2_b, c3_w, c3_b, c4_w, c4_b,
           W5, b5, W6, b6, W7, b7, W8, b8, W9, b9, W10, b10):
    bf = jnp.bfloat16
    x = batch[0].transpose(1, 2, 0).astype(bf)
    x = _conv(x, _w(c1_w.transpose(2, 3, 1, 0).reshape(9, 3, 64)), c1_b, th=0, edge=False)
    x = _conv(x, _w(c2_w.transpose(2, 3, 1, 0).reshape(9, 64, 64)), c2_b, th=112, edge=False)
    x = _pool(x, th2=16)
    x = _conv(x, _w(c3_w.transpose(2, 3, 1, 0).reshape(9, 64, 128)), c3_b, th=0, edge=False)
    x = _conv(x, _w(c4_w.transpose(2, 3, 1, 0).reshape(9, 128, 128)), c4_b, th=0, edge=False)
    x = _pool(x, th2=56)
    x = _conv(x, _w(W5.reshape(9, 128, 256)), b5, th=0, edge=True, wp=64)
    x = _conv(x, _w(W6.reshape(9, 256, 256)), b6, th=0, edge=True, wp=64)
    x = _conv(x, _w(W7.reshape(9, 256, 256)), b7, th=0, edge=True, wp=64)
    x = _pool(x, th2=28)
    x = _conv(x, _w(W8.reshape(9, 256, 512)), b8, th=0, edge=True, wp=32)
    x = _conv(x, _w(W9.reshape(9, 512, 512)), b9, th=0, edge=True, wp=32)
    x = _conv(x, _w(W10.reshape(9, 512, 512)), b10, th=0, edge=True, wp=32,
              out_dtype=jnp.float32)
    return x.transpose(2, 0, 1)[None]


# speedup vs baseline: 4.1367x; 1.1907x over previous
import functools

import jax
import jax.numpy as jnp
from jax.experimental import pallas as pl
from jax.experimental.pallas import tpu as pltpu

_HEAD = 16


def _conv_body(x_ref, w_ref, b_ref, o_ref, *, m, wp, out_dtype):
    acc = None
    for dh in range(3):
        s = x_ref[0, pl.ds(dh * wp, m), :]
        p = jnp.dot(s, w_ref[dh], preferred_element_type=jnp.float32)
        acc = p if acc is None else acc + p
    o_ref[0] = jnp.maximum(acc + b_ref[...], 0.0).astype(out_dtype)


def _conv(x, wt, b, th, wp=None, out_dtype=jnp.bfloat16):
    h, w, cin = x.shape
    cout = wt.shape[2]
    wp = wp or w
    th = th or h
    nt = h // th
    m = th * wp
    xp = jnp.pad(x, ((1, 1), (1, 1), (0, 0)))
    xcat = jnp.concatenate([xp[:, d:d + w, :] for d in range(3)], axis=2)
    if wp > w:
        xcat = jnp.pad(xcat, ((0, 0), (0, wp - w), (0, 0)))
    xt = jnp.stack([jax.lax.slice_in_dim(xcat, i * th, i * th + th + 2, axis=0)
                    for i in range(nt)]).reshape(nt, (th + 2) * wp, 3 * cin)
    out = pl.pallas_call(
        functools.partial(_conv_body, m=m, wp=wp, out_dtype=out_dtype),
        grid=(nt,),
        in_specs=[
            pl.BlockSpec((1, (th + 2) * wp, 3 * cin), lambda i: (i, 0, 0)),
            pl.BlockSpec((3, 3 * cin, cout), lambda i: (0, 0, 0)),
            pl.BlockSpec((1, cout), lambda i: (0, 0)),
        ],
        out_specs=pl.BlockSpec((1, m, cout), lambda i: (i, 0, 0)),
        out_shape=jax.ShapeDtypeStruct((nt, m, cout), out_dtype),
    )(xt, wt, b.reshape(1, cout))
    return out.reshape(h, wp, cout)[:, :w, :]


def _group_body(*refs, nl, h, w, wp, edge, out_dtype):
    m = h * wp
    views, wbs = refs[:4], refs[4:4 + 2 * nl]
    o_ref = refs[4 + 2 * nl]
    pfs = refs[5 + 2 * nl:]
    jcol = jax.lax.broadcasted_iota(jnp.int32, (m, 1), 0) & (wp - 1)
    cur = jnp.maximum(jnp.maximum(views[0][...], views[1][...]),
                      jnp.maximum(views[2][...], views[3][...]))
    for l in range(nl):
        pf = pfs[l]
        if not edge:
            pf[...] = jnp.zeros(pf.shape, pf.dtype)
        pf[pl.ds(_HEAD + wp, m), :] = cur.astype(jnp.bfloat16)
        if edge:
            pf[pl.ds(_HEAD, wp), :] = pf[pl.ds(_HEAD + wp, wp), :]
            pf[pl.ds(_HEAD + (h + 1) * wp, wp), :] = pf[pl.ds(_HEAD + h * wp, wp), :]
        w_ref, b_ref = wbs[2 * l], wbs[2 * l + 1]
        acc = None
        for dh in range(3):
            base = _HEAD - 1 + dh * wp
            s1 = pf[pl.ds(base + 1, m), :]
            s0 = pf[pl.ds(base, m), :]
            s2 = pf[pl.ds(base + 2, m), :]
            if edge:
                s0 = jnp.where(jcol == 0, s1, s0)
                s2 = jnp.where(jcol == w - 1, s1, s2)
            for dw, s in ((0, s0), (1, s1), (2, s2)):
                p = jnp.dot(s, w_ref[dh * 3 + dw], preferred_element_type=jnp.float32)
                acc = p if acc is None else acc + p
        act = jnp.maximum(acc + b_ref[...], 0.0)
        if l == nl - 1:
            o_ref[...] = act.astype(out_dtype)
        else:
            cur = act if edge else jnp.where(jcol < w, act, 0.0)


def _group(x, wbs, wp, edge, out_dtype=jnp.bfloat16):
    h2, w2, c0 = x.shape[0] // 2, x.shape[1] // 2, x.shape[2]
    m = h2 * wp
    views = [jnp.pad(x[i::2, j::2, :], ((0, 0), (0, wp - w2), (0, 0))).reshape(m, c0)
             for i in (0, 1) for j in (0, 1)]
    nl = len(wbs)
    cout = wbs[-1][0].shape[2]
    cins = [wb[0].shape[1] for wb in wbs]
    args = []
    for wt, b in wbs:
        args += [wt.astype(jnp.bfloat16), b.reshape(1, -1)]
    out = pl.pallas_call(
        functools.partial(_group_body, nl=nl, h=h2, w=w2, wp=wp, edge=edge,
                          out_dtype=out_dtype),
        out_shape=jax.ShapeDtypeStruct((m, cout), out_dtype),
        scratch_shapes=[pltpu.VMEM((m + 2 * wp + 2 * _HEAD, c), jnp.bfloat16)
                        for c in cins],
    )(*views, *args)
    return out.reshape(h2, wp, cout)[:, :w2, :]


def _w3(wt):
    return wt.reshape(3, -1, wt.shape[2]).astype(jnp.bfloat16)


def kernel(batch, pooling_mask, c1_w, c1_b, c2_w, c2_b, c3_w, c3_b, c4_w, c4_b,
           W5, b5, W6, b6, W7, b7, W8, b8, W9, b9, W10, b10):
    x = batch[0].transpose(1, 2, 0).astype(jnp.bfloat16)
    x = _conv(x, _w3(c1_w.transpose(2, 3, 1, 0).reshape(9, 3, 64)), c1_b, th=0)
    x = _conv(x, _w3(c2_w.transpose(2, 3, 1, 0).reshape(9, 64, 64)), c2_b, th=112)
    x = _group(x, [(c3_w.transpose(2, 3, 1, 0).reshape(9, 64, 128), c3_b),
                   (c4_w.transpose(2, 3, 1, 0).reshape(9, 128, 128), c4_b)],
               wp=128, edge=False)
    x = _group(x, [(W5.reshape(9, 128, 256), b5),
                   (W6.reshape(9, 256, 256), b6),
                   (W7.reshape(9, 256, 256), b7)],
               wp=64, edge=True)
    x = _group(x, [(W8.reshape(9, 256, 512), b8),
                   (W9.reshape(9, 512, 512), b9),
                   (W10.reshape(9, 512, 512), b10)],
               wp=32, edge=True, out_dtype=jnp.float32)
    return x.transpose(2, 0, 1)[None]


# speedup vs baseline: 4.5279x; 1.0946x over previous
import functools

import jax
import jax.numpy as jnp
from jax.experimental import pallas as pl
from jax.experimental.pallas import tpu as pltpu

_HEAD = 16


def _conv1_body(x_ref, w_ref, b_ref, o_ref, *, m, wp, w, ns):
    ms = m // ns
    cout = o_ref.shape[1]
    jc = jax.lax.broadcasted_iota(jnp.int32, (ms, cout), 0) & (wp - 1)
    for s in range(ns):
        soff = s * ms
        acc = None
        for dh in range(3):
            x = x_ref[pl.ds(dh * wp + soff, ms), :]
            p = jnp.dot(x, w_ref[dh], preferred_element_type=jnp.float32)
            acc = p if acc is None else acc + p
        act = jnp.maximum(acc + b_ref[...], 0.0)
        act = jnp.where(jc < w, act, 0.0)
        o_ref[pl.ds(soff, ms), :] = act.astype(jnp.bfloat16)


def _conv1(x, wt, b, wp, ns):
    h, w, cin = x.shape
    cout = wt.shape[2]
    m = h * wp
    xp = jnp.pad(x, ((1, 1), (1, 1), (0, 0)))
    xcat = jnp.concatenate([xp[:, d:d + w, :] for d in range(3)], axis=2)
    xcat = jnp.pad(xcat, ((0, 0), (0, wp - w), (0, 0))).reshape((h + 2) * wp, 3 * cin)
    return pl.pallas_call(
        functools.partial(_conv1_body, m=m, wp=wp, w=w, ns=ns),
        out_shape=jax.ShapeDtypeStruct((m, cout), jnp.bfloat16),
    )(xcat, wt, b.reshape(1, cout))


def _group_body(*refs, nv, nl, h, w, wp, edge, ns, dma_in, out_dtype):
    m = h * wp
    ms = m // ns
    views, wbs = refs[:nv], refs[nv:nv + 2 * nl]
    o_ref = refs[nv + 2 * nl]
    pfs = refs[nv + 1 + 2 * nl:]
    def jc(c):
        return jax.lax.broadcasted_iota(jnp.int32, (ms, c), 0) & (wp - 1)
    if dma_in:
        sem = pfs[-1]
        pfs = pfs[:-1]
    if not edge:
        pfs[0][...] = jnp.zeros(pfs[0].shape, pfs[0].dtype)
    if dma_in:
        cp = pltpu.make_async_copy(views[0], pfs[0].at[pl.ds(_HEAD + wp, m), :], sem)
        cp.start()
        cp.wait()
    else:
        if nv == 4:
            cur = jnp.maximum(jnp.maximum(views[0][...], views[1][...]),
                              jnp.maximum(views[2][...], views[3][...]))
        else:
            cur = views[0][...]
        pfs[0][pl.ds(_HEAD + wp, m), :] = cur.astype(jnp.bfloat16)
    for l in range(nl):
        pf = pfs[l]
        if edge:
            pf[pl.ds(_HEAD, wp), :] = pf[pl.ds(_HEAD + wp, wp), :]
            pf[pl.ds(_HEAD + (h + 1) * wp, wp), :] = pf[pl.ds(_HEAD + h * wp, wp), :]
        elif l + 1 < nl:
            pfs[l + 1][...] = jnp.zeros(pfs[l + 1].shape, pfs[l + 1].dtype)
        w_ref, b_ref = wbs[2 * l], wbs[2 * l + 1]
        cin = pf.shape[1]
        jci = jc(cin) if edge else None
        for s in range(ns):
            soff = s * ms
            acc = None
            for dh in range(3):
                base = _HEAD - 1 + dh * wp + soff
                s1 = pf[pl.ds(base + 1, ms), :]
                s0 = pf[pl.ds(base, ms), :]
                s2 = pf[pl.ds(base + 2, ms), :]
                if edge:
                    s0 = jnp.where(jci == 0, s1, s0)
                    s2 = jnp.where(jci == w - 1, s1, s2)
                for dw, sv in ((0, s0), (1, s1), (2, s2)):
                    p = jnp.dot(sv, w_ref[dh * 3 + dw], preferred_element_type=jnp.float32)
                    acc = p if acc is None else acc + p
            act = jnp.maximum(acc + b_ref[...], 0.0)
            if l == nl - 1:
                o_ref[pl.ds(soff, ms), :] = act.astype(out_dtype)
            else:
                if not edge:
                    act = jnp.where(jc(act.shape[1]) < w, act, 0.0)
                pfs[l + 1][pl.ds(_HEAD + wp + soff, ms), :] = act.astype(jnp.bfloat16)


def _group(views, wbs, h, w, wp, edge, ns=1, dma_in=False, out_dtype=jnp.bfloat16):
    m = h * wp
    nl = len(wbs)
    cout = wbs[-1][0].shape[2]
    cins = [wb[0].shape[1] for wb in wbs]
    args = []
    for wt, b in wbs:
        args += [wt.astype(jnp.bfloat16), b.reshape(1, -1)]
    scratch = [pltpu.VMEM((m + 2 * wp + 2 * _HEAD, c), jnp.bfloat16) for c in cins]
    in_specs = [pl.BlockSpec(memory_space=pltpu.MemorySpace.HBM) if dma_in else pl.BlockSpec()
                for _ in views] + [pl.BlockSpec() for _ in args]
    if dma_in:
        scratch = scratch + [pltpu.SemaphoreType.DMA]
    return pl.pallas_call(
        functools.partial(_group_body, nv=len(views), nl=nl, h=h, w=w, wp=wp,
                          edge=edge, ns=ns, dma_in=dma_in, out_dtype=out_dtype),
        in_specs=in_specs,
        out_shape=jax.ShapeDtypeStruct((m, cout), out_dtype),
        scratch_shapes=scratch,
    )(*views, *args)


def _pool_views(a, h, w, wp_in, wp_out):
    c = a.shape[1]
    x3 = a.reshape(h, wp_in, c)[:, :w, :]
    h2, w2 = h // 2, w // 2
    return [jnp.pad(x3[i::2, j::2, :], ((0, 0), (0, wp_out - w2), (0, 0)))
            .reshape(h2 * wp_out, c) for i in (0, 1) for j in (0, 1)]


def _t9(cw):
    return cw.transpose(2, 3, 1, 0).reshape(9, cw.shape[1], cw.shape[0])


def kernel(batch, pooling_mask, c1_w, c1_b, c2_w, c2_b, c3_w, c3_b, c4_w, c4_b,
           W5, b5, W6, b6, W7, b7, W8, b8, W9, b9, W10, b10):
    x = batch[0].transpose(1, 2, 0).astype(jnp.bfloat16)
    a = _conv1(x, _t9(c1_w).reshape(3, 9, 64).astype(jnp.bfloat16), c1_b,
               wp=256, ns=2)
    a = _group([a], [(_t9(c2_w), c2_b)], h=224, w=224, wp=256, edge=False, ns=4, dma_in=True)
    a = _group(_pool_views(a, 224, 224, 256, 128),
               [(_t9(c3_w), c3_b), (_t9(c4_w), c4_b)],
               h=112, w=112, wp=128, edge=False)
    a = _group(_pool_views(a, 112, 112, 128, 64),
               [(W5.reshape(9, 128, 256), b5),
                (W6.reshape(9, 256, 256), b6),
                (W7.reshape(9, 256, 256), b7)],
               h=56, w=56, wp=64, edge=True)
    a = _group(_pool_views(a, 56, 56, 64, 32),
               [(W8.reshape(9, 256, 512), b8),
                (W9.reshape(9, 512, 512), b9),
                (W10.reshape(9, 512, 512), b10)],
               h=28, w=28, wp=32, edge=True, out_dtype=jnp.float32)
    return a.reshape(28, 32, 512)[:, :28, :].transpose(2, 0, 1)[None]
